# baseline (speedup 1.0000x reference)
import jax
import jax.numpy as jnp
from jax.experimental import pallas as pl


def _tok_kernel(flo_ref, fhi_ref, wn_ref, wr_ref, cn_ref, cr_ref, pos_ref, out_ref):
    p_half = pos_ref.shape[0]
    lanes = pos_ref.shape[1]
    d = lanes // 2
    base_n = pos_ref[...] + cn_ref[0, :][None, :]
    base_r = pos_ref[...] + cr_ref[0, :][None, :]
    lane = jax.lax.broadcasted_iota(jnp.int32, (1, 1, lanes), 2)
    is_lo = lane < d

    def half(f_lo, f_hi, w, base):
        tb = f_lo.shape[0]
        fb_lo = jnp.broadcast_to(f_lo[:, :, None], (tb, p_half, lanes))
        fb_hi = jnp.broadcast_to(f_hi[:, :, None], (tb, p_half, lanes))
        f_sel = jnp.where(is_lo, fb_lo, fb_hi)
        return f_sel * w[0, :][None, None, :] + base[None, :, :]

    out_ref[:, :p_half, :] = half(flo_ref[:, :p_half], fhi_ref[:, :p_half],
                                  wn_ref, base_n)
    out_ref[:, p_half:, :] = half(flo_ref[:, p_half:], fhi_ref[:, p_half:],
                                  wr_ref, base_r)


def kernel(features_nir, features_raman, W_nir, b_nir, W_raman, b_raman, pos_table, spec_table):
    B, N = features_nir.shape
    D = pos_table.shape[1]
    TB = 32
    P = N

    f_all = jnp.concatenate([features_nir, features_raman], axis=1)
    f3 = f_all.reshape(B, P, 2)
    f_lo = f3[:, :, 0]
    f_hi = f3[:, :, 1]

    pos2 = pos_table.reshape(N // 2, 2 * D)
    w_n2 = jnp.tile(W_nir[:, 0], 2)[None, :]
    w_r2 = jnp.tile(W_raman[:, 0], 2)[None, :]
    c_n2 = jnp.tile(b_nir + spec_table[0], 2)[None, :]
    c_r2 = jnp.tile(b_raman + spec_table[1], 2)[None, :]

    grid = (B // TB,)
    out = pl.pallas_call(
        _tok_kernel,
        grid=grid,
        in_specs=[
            pl.BlockSpec((TB, P), lambda i: (i, 0)),
            pl.BlockSpec((TB, P), lambda i: (i, 0)),
            pl.BlockSpec((1, 2 * D), lambda i: (0, 0)),
            pl.BlockSpec((1, 2 * D), lambda i: (0, 0)),
            pl.BlockSpec((1, 2 * D), lambda i: (0, 0)),
            pl.BlockSpec((1, 2 * D), lambda i: (0, 0)),
            pl.BlockSpec((N // 2, 2 * D), lambda i: (0, 0)),
        ],
        out_specs=pl.BlockSpec((TB, P, 2 * D), lambda i: (i, 0, 0)),
        out_shape=jax.ShapeDtypeStruct((B, P, 2 * D), features_nir.dtype),
    )(f_lo, f_hi, w_n2, w_r2, c_n2, c_r2, pos2)
    return out.reshape(B, 2 * N, D)


# speedup vs baseline: 6.9513x; 6.9513x over previous
import jax
import jax.numpy as jnp
from jax.experimental import pallas as pl


def _tok_kernel(fn_ref, fr_ref, wn_ref, wr_ref, cn_ref, cr_ref, pos_ref, out_ref):
    d, n = pos_ref.shape
    tb = fn_ref.shape[0]
    base_n = pos_ref[...] + cn_ref[:, 0][:, None]
    base_r = pos_ref[...] + cr_ref[:, 0][:, None]
    f_n = jnp.broadcast_to(fn_ref[...][:, None, :], (tb, d, n))
    f_r = jnp.broadcast_to(fr_ref[...][:, None, :], (tb, d, n))
    out_ref[:, :, :n] = f_n * wn_ref[:, 0][None, :, None] + base_n[None, :, :]
    out_ref[:, :, n:] = f_r * wr_ref[:, 0][None, :, None] + base_r[None, :, :]


def kernel(features_nir, features_raman, W_nir, b_nir, W_raman, b_raman, pos_table, spec_table):
    B, N = features_nir.shape
    D = pos_table.shape[1]
    TB = 32

    pos_t = pos_table.T
    c_n = (b_nir + spec_table[0])[:, None]
    c_r = (b_raman + spec_table[1])[:, None]

    grid = (B // TB,)
    out_t = pl.pallas_call(
        _tok_kernel,
        grid=grid,
        in_specs=[
            pl.BlockSpec((TB, N), lambda i: (i, 0)),
            pl.BlockSpec((TB, N), lambda i: (i, 0)),
            pl.BlockSpec((D, 1), lambda i: (0, 0)),
            pl.BlockSpec((D, 1), lambda i: (0, 0)),
            pl.BlockSpec((D, 1), lambda i: (0, 0)),
            pl.BlockSpec((D, 1), lambda i: (0, 0)),
            pl.BlockSpec((D, N), lambda i: (0, 0)),
        ],
        out_specs=pl.BlockSpec((TB, D, 2 * N), lambda i: (i, 0, 0)),
        out_shape=jax.ShapeDtypeStruct((B, D, 2 * N), features_nir.dtype),
    )(features_nir, features_raman, W_nir, W_raman, c_n, c_r, pos_t)
    return out_t.transpose(0, 2, 1)
